# Initial kernel scaffold; baseline (speedup 1.0000x reference)
import functools

import jax
import jax.numpy as jnp
from jax import lax
from jax.experimental import pallas as pl
from jax.experimental.pallas import tpu as pltpu
from jax.experimental.pallas import tpu_sc as plsc

N = 10000
E = 320000
NP = 10240
NC = 2
NS = 16
EK = 80
EV = E // (NC * NS)
ED = E // NS
RPT = NP // NS
BR = 512

_MESH = plsc.VectorSubcoreMesh(
    core_axis_name="c", subcore_axis_name="s", num_cores=NC, num_subcores=NS)


def _fisr(d):
    i = plsc.bitcast(d, jnp.int32)
    i = jnp.int32(0x5F3759DF) - (i >> 1)
    y = plsc.bitcast(i, jnp.float32)
    for _ in range(3):
        y = y * (1.5 - 0.5 * d * y * y)
    return y


@functools.partial(
    pl.kernel,
    out_type=jax.ShapeDtypeStruct((NP,), jnp.float32),
    mesh=_MESH,
    scratch_types=[
        pltpu.VMEM((EK,), jnp.int32),
        pltpu.VMEM((EK,), jnp.float32),
        pltpu.VMEM((RPT,), jnp.float32),
        pltpu.VMEM((NP // (NC * NS),), jnp.float32),
        pltpu.VMEM_SHARED((NP,), jnp.float32),
    ],
)
def _deg_dis(dst_hbm, dis_hbm, didx, ones, dbuf, disbuf, dacc):
    cid = lax.axis_index("c")
    tid = lax.axis_index("s")

    def fill(i, _):
        dbuf[pl.ds(i * 16, 16)] = jnp.zeros((16,), jnp.float32)
        return 0
    lax.fori_loop(0, RPT // 16, fill, 0)

    def fill1(i, _):
        ones[pl.ds(i * 16, 16)] = jnp.ones((16,), jnp.float32)
        return 0
    lax.fori_loop(0, EK // 16, fill1, 0)

    pltpu.sync_copy(dbuf, dacc.at[pl.ds(tid * RPT, RPT)])
    plsc.subcore_barrier()

    def body(j, _):
        base = tid * ED + j * EK
        pltpu.sync_copy(dst_hbm.at[pl.ds(base, EK)], didx)
        pltpu.sync_copy(ones, dacc.at[didx], add=True)
        return 0
    lax.fori_loop(0, ED // EK, body, 0)
    plsc.subcore_barrier()

    nr = NP // (NC * NS)
    r0 = (cid * NS + tid) * nr
    pltpu.sync_copy(dacc.at[pl.ds(r0, nr)], dbuf.at[pl.ds(0, nr)])

    def conv(i, _):
        d = dbuf[pl.ds(i * 16, 16)] + 1.0
        disbuf[pl.ds(i * 16, 16)] = _fisr(d)
        return 0
    lax.fori_loop(0, nr // 16, conv, 0)
    pltpu.sync_copy(disbuf, dis_hbm.at[pl.ds(r0, nr)])


@functools.partial(
    pl.kernel,
    out_type=jax.ShapeDtypeStruct((NC, NP, 128), jnp.float32),
    mesh=_MESH,
    scratch_types=[
        pltpu.VMEM((EK,), jnp.int32),
        pltpu.VMEM((EK,), jnp.int32),
        pltpu.VMEM((EK, 128), jnp.float32),
        pltpu.VMEM((128, 128), jnp.float32),
        pltpu.SemaphoreType.DMA,
        pltpu.VMEM_SHARED((NP, 128), jnp.float32),
    ],
)
def _agg(u_hbm, src_hbm, dst_hbm, out_hbm, sidx, didx, rows, zblk, sem, acc):
    cid = lax.axis_index("c")
    tid = lax.axis_index("s")
    wid = tid * NC + cid

    def fill(i, _):
        zblk[i // 8, pl.ds((i % 8) * 16, 16)] = jnp.zeros((16,), jnp.float32)
        return 0
    lax.fori_loop(0, 128 * 8, fill, 0)

    r0 = tid * RPT
    for k in range(RPT // 128):
        pltpu.sync_copy(zblk, acc.at[pl.ds(r0 + k * 128, 128)])
    plsc.subcore_barrier()

    ebase = wid * EV

    def body(j, _):
        b = ebase + j * EK
        pltpu.sync_copy(src_hbm.at[pl.ds(b, EK)], sidx)
        pltpu.sync_copy(dst_hbm.at[pl.ds(b, EK)], didx)
        pltpu.async_copy(u_hbm.at[sidx], rows, sem).wait()
        pltpu.sync_copy(rows, acc.at[didx], add=True)
        return 0
    lax.fori_loop(0, EV // EK, body, 0)
    plsc.subcore_barrier()

    for k in range(RPT // 128):
        pltpu.sync_copy(acc.at[pl.ds(r0 + k * 128, 128)],
                        out_hbm.at[cid, pl.ds(r0 + k * 128, 128)])


def _scale_body(x_ref, d_ref, o_ref):
    o_ref[...] = x_ref[...] * d_ref[...]


_scale = pl.pallas_call(
    _scale_body,
    grid=(NP // BR,),
    in_specs=[
        pl.BlockSpec((BR, 128), lambda i: (i, 0)),
        pl.BlockSpec((BR, 1), lambda i: (i, 0)),
    ],
    out_specs=pl.BlockSpec((BR, 128), lambda i: (i, 0)),
    out_shape=jax.ShapeDtypeStruct((NP, 128), jnp.float32),
)


def _mid_body(p_ref, u_ref, d_ref, w1_ref, b1_ref, w2_ref, o_ref):
    a = (u_ref[...] + p_ref[0] + p_ref[1]) * d_ref[...]
    y = jnp.dot(a, w1_ref[...], preferred_element_type=jnp.float32)
    y = jnp.maximum(y + b1_ref[...], 0.0)
    o = jnp.dot(y, w2_ref[...], preferred_element_type=jnp.float32)
    o_ref[...] = o * d_ref[...]


_mid = pl.pallas_call(
    _mid_body,
    grid=(NP // BR,),
    in_specs=[
        pl.BlockSpec((NC, BR, 128), lambda i: (0, i, 0)),
        pl.BlockSpec((BR, 128), lambda i: (i, 0)),
        pl.BlockSpec((BR, 1), lambda i: (i, 0)),
        pl.BlockSpec((128, 256), lambda i: (0, 0)),
        pl.BlockSpec((1, 256), lambda i: (0, 0)),
        pl.BlockSpec((256, 128), lambda i: (0, 0)),
    ],
    out_specs=pl.BlockSpec((BR, 128), lambda i: (i, 0)),
    out_shape=jax.ShapeDtypeStruct((NP, 128), jnp.float32),
)


def _fin_body(p_ref, u_ref, d_ref, b2_ref, o_ref):
    s = (u_ref[...] + p_ref[0] + p_ref[1]) * d_ref[...]
    o_ref[...] = jnp.maximum(s + b2_ref[...], 0.0)


_fin = pl.pallas_call(
    _fin_body,
    grid=(NP // BR,),
    in_specs=[
        pl.BlockSpec((NC, BR, 128), lambda i: (0, i, 0)),
        pl.BlockSpec((BR, 128), lambda i: (i, 0)),
        pl.BlockSpec((BR, 1), lambda i: (i, 0)),
        pl.BlockSpec((1, 128), lambda i: (0, 0)),
    ],
    out_specs=pl.BlockSpec((BR, 128), lambda i: (i, 0)),
    out_shape=jax.ShapeDtypeStruct((NP, 128), jnp.float32),
)


def kernel(x, edge_index, W1, b1, W2, b2):
    ei = edge_index.astype(jnp.int32)
    src = ei[0]
    dst = ei[1]
    xp = jnp.pad(x, ((0, NP - N), (0, 0)))

    dis = _deg_dis(dst)
    dis2 = dis.reshape(NP, 1)
    u1 = _scale(xp, dis2)
    p1 = _agg(u1, src, dst)
    u2 = _mid(p1, u1, dis2, W1, b1.reshape(1, -1), W2)
    p2 = _agg(u2, src, dst)
    y = _fin(p2, u2, dis2, b2.reshape(1, -1))
    return y[:N]


# speedup vs baseline: 13.2144x; 13.2144x over previous
import functools

import jax
import jax.numpy as jnp
from jax import lax
from jax.experimental import pallas as pl
from jax.experimental.pallas import tpu as pltpu
from jax.experimental.pallas import tpu_sc as plsc

N = 10000
E = 320000
NP = 10240
NC = 2
NS = 16
EK = 80
EV = E // (NC * NS)
ED = E // NS
RPT = NP // NS
BR = 512

def _mesh():
    return plsc.VectorSubcoreMesh(
        core_axis_name="c", subcore_axis_name="s",
        num_cores=NC, num_subcores=NS)


@functools.cache
def _make_deg():
    return functools.partial(
        pl.kernel,
        out_type=jax.ShapeDtypeStruct((NP,), jnp.float32),
        mesh=_mesh(),
        scratch_types=[
            pltpu.VMEM((EK,), jnp.int32),
            pltpu.VMEM((EK,), jnp.float32),
            pltpu.VMEM((RPT,), jnp.float32),
            pltpu.VMEM_SHARED((NP,), jnp.float32),
        ],
    )(_deg_body)


def _deg_body(dst_hbm, deg_hbm, didx, ones, dbuf, dacc):
    cid = lax.axis_index("c")
    tid = lax.axis_index("s")

    def fill(i, _):
        dbuf[pl.ds(i * 16, 16)] = jnp.zeros((16,), jnp.float32)
        return 0
    lax.fori_loop(0, RPT // 16, fill, 0)

    def fill1(i, _):
        ones[pl.ds(i * 16, 16)] = jnp.ones((16,), jnp.float32)
        return 0
    lax.fori_loop(0, EK // 16, fill1, 0)

    pltpu.sync_copy(dbuf, dacc.at[pl.ds(tid * RPT, RPT)])
    plsc.subcore_barrier()

    def body(j, _):
        base = tid * ED + j * EK
        pltpu.sync_copy(dst_hbm.at[pl.ds(base, EK)], didx)
        pltpu.sync_copy(ones, dacc.at[didx], add=True)
        return 0
    lax.fori_loop(0, ED // EK, body, 0)
    plsc.subcore_barrier()

    nr = NP // (NC * NS)
    r0 = (cid * NS + tid) * nr
    pltpu.sync_copy(dacc.at[pl.ds(r0, nr)], dbuf.at[pl.ds(0, nr)])
    pltpu.sync_copy(dbuf.at[pl.ds(0, nr)], deg_hbm.at[pl.ds(r0, nr)])


@functools.cache
def _make_agg():
    return functools.partial(
        pl.kernel,
        out_type=jax.ShapeDtypeStruct((NC, NP, 128), jnp.float32),
        mesh=_mesh(),
        scratch_types=[
            pltpu.VMEM((EK,), jnp.int32),
            pltpu.VMEM((EK,), jnp.int32),
            pltpu.VMEM((EK, 128), jnp.float32),
            pltpu.VMEM((128, 128), jnp.float32),
            pltpu.SemaphoreType.DMA,
            pltpu.VMEM_SHARED((NP, 128), jnp.float32),
        ],
    )(_agg_body)


def _agg_body(u_hbm, src_hbm, dst_hbm, out_hbm, sidx, didx, rows, zblk, sem, acc):
    cid = lax.axis_index("c")
    tid = lax.axis_index("s")
    wid = tid * NC + cid

    def fill(i, _):
        zblk[i // 8, pl.ds((i % 8) * 16, 16)] = jnp.zeros((16,), jnp.float32)
        return 0
    lax.fori_loop(0, 128 * 8, fill, 0)

    r0 = tid * RPT
    for k in range(RPT // 128):
        pltpu.sync_copy(zblk, acc.at[pl.ds(r0 + k * 128, 128)])
    plsc.subcore_barrier()

    ebase = wid * EV

    def body(j, _):
        b = ebase + j * EK
        pltpu.sync_copy(src_hbm.at[pl.ds(b, EK)], sidx)
        pltpu.sync_copy(dst_hbm.at[pl.ds(b, EK)], didx)
        pltpu.async_copy(u_hbm.at[sidx], rows, sem).wait()
        pltpu.sync_copy(rows, acc.at[didx], add=True)
        return 0
    lax.fori_loop(0, EV // EK, body, 0)
    plsc.subcore_barrier()

    for k in range(RPT // 128):
        pltpu.sync_copy(acc.at[pl.ds(r0 + k * 128, 128)], zblk)
        pltpu.sync_copy(zblk, out_hbm.at[cid, pl.ds(r0 + k * 128, 128)])


def _scale_body(x_ref, deg_ref, u_ref, dis_ref):
    dis = lax.rsqrt(deg_ref[...] + 1.0)
    dis_ref[...] = dis
    u_ref[...] = x_ref[...] * dis


_scale = pl.pallas_call(
    _scale_body,
    grid=(NP // BR,),
    in_specs=[
        pl.BlockSpec((BR, 128), lambda i: (i, 0)),
        pl.BlockSpec((BR, 1), lambda i: (i, 0)),
    ],
    out_specs=[
        pl.BlockSpec((BR, 128), lambda i: (i, 0)),
        pl.BlockSpec((BR, 1), lambda i: (i, 0)),
    ],
    out_shape=[
        jax.ShapeDtypeStruct((NP, 128), jnp.float32),
        jax.ShapeDtypeStruct((NP, 1), jnp.float32),
    ],
)


def _mid_body(p_ref, u_ref, d_ref, w1_ref, b1_ref, w2_ref, o_ref):
    a = (u_ref[...] + p_ref[0] + p_ref[1]) * d_ref[...]
    y = jnp.dot(a, w1_ref[...], preferred_element_type=jnp.float32)
    y = jnp.maximum(y + b1_ref[...], 0.0)
    o = jnp.dot(y, w2_ref[...], preferred_element_type=jnp.float32)
    o_ref[...] = o * d_ref[...]


_mid = pl.pallas_call(
    _mid_body,
    grid=(NP // BR,),
    in_specs=[
        pl.BlockSpec((NC, BR, 128), lambda i: (0, i, 0)),
        pl.BlockSpec((BR, 128), lambda i: (i, 0)),
        pl.BlockSpec((BR, 1), lambda i: (i, 0)),
        pl.BlockSpec((128, 256), lambda i: (0, 0)),
        pl.BlockSpec((1, 256), lambda i: (0, 0)),
        pl.BlockSpec((256, 128), lambda i: (0, 0)),
    ],
    out_specs=pl.BlockSpec((BR, 128), lambda i: (i, 0)),
    out_shape=jax.ShapeDtypeStruct((NP, 128), jnp.float32),
)


def _fin_body(p_ref, u_ref, d_ref, b2_ref, o_ref):
    s = (u_ref[...] + p_ref[0] + p_ref[1]) * d_ref[...]
    o_ref[...] = jnp.maximum(s + b2_ref[...], 0.0)


_fin = pl.pallas_call(
    _fin_body,
    grid=(NP // BR,),
    in_specs=[
        pl.BlockSpec((NC, BR, 128), lambda i: (0, i, 0)),
        pl.BlockSpec((BR, 128), lambda i: (i, 0)),
        pl.BlockSpec((BR, 1), lambda i: (i, 0)),
        pl.BlockSpec((1, 128), lambda i: (0, 0)),
    ],
    out_specs=pl.BlockSpec((BR, 128), lambda i: (i, 0)),
    out_shape=jax.ShapeDtypeStruct((NP, 128), jnp.float32),
)


def kernel(x, edge_index, W1, b1, W2, b2):
    ei = edge_index.astype(jnp.int32)
    src = ei[0]
    dst = ei[1]
    xp = jnp.pad(x, ((0, NP - N), (0, 0)))

    agg = _make_agg()
    deg = _make_deg()(dst)
    u1, dis2 = _scale(xp, deg.reshape(NP, 1))
    p1 = agg(u1, src, dst)
    u2 = _mid(p1, u1, dis2, W1, b1.reshape(1, -1), W2)
    p2 = agg(u2, src, dst)
    y = _fin(p2, u2, dis2, b2.reshape(1, -1))
    return y[:N]
